# baseline (speedup 1.0000x reference)
import functools

import jax
import jax.numpy as jnp
import numpy as np
from jax import lax
from jax.experimental import pallas as pl
from jax.experimental.pallas import tpu as pltpu
from jax.experimental.pallas import tpu_sc as plsc

VOCAB = 100000
D_MODEL = 768
B = 4
S = 2048

_NC = 2
_NS = 16
_NW = _NC * _NS

_SPW = S // _NW
_LANES = 16
_VPR = D_MODEL // _LANES


def _sinusoidal_pe(max_len, d_model):
    pos = np.arange(max_len, dtype=np.float64)[:, None]
    div = np.exp(
        np.arange(0, d_model, 2, dtype=np.float64) * -(np.log(10000.0) / d_model)
    )
    pe = np.zeros((max_len, d_model), dtype=np.float64)
    pe[:, 0::2] = np.sin(pos * div)
    pe[:, 1::2] = np.cos(pos * div)
    return pe.astype(np.float32)


_PE = _sinusoidal_pe(S, D_MODEL)


_HALF = _SPW // 2
_NCHUNK = 2 * B


def _sc_body(table_hbm, idx_hbm, pe_hbm, out_hbm,
             idx_v, pe_v, rows_a, rows_b, gs_a, gs_b, os_a, os_b):
    wid = lax.axis_index("s") * _NC + lax.axis_index("c")
    s0 = wid * _SPW

    bufs = (rows_a, rows_b)
    gsems = (gs_a, gs_b)
    osems = (os_a, os_b)

    for b in range(B):
        pltpu.sync_copy(idx_hbm.at[b, pl.ds(s0, _SPW)],
                        idx_v.at[pl.ds(b * _SPW, _SPW)])

    def start_gather(k):
        return pltpu.async_copy(
            table_hbm.at[idx_v.at[pl.ds(k * _HALF, _HALF)]],
            bufs[k % 2], gsems[k % 2])

    gath = [None] * _NCHUNK
    outc = [None] * _NCHUNK
    gath[0] = start_gather(0)
    pltpu.sync_copy(pe_hbm.at[pl.ds(s0, _SPW)], pe_v)

    for k in range(_NCHUNK):
        if k >= 1:
            outc[k - 1].wait()
        if k + 1 < _NCHUNK:
            gath[k + 1] = start_gather(k + 1)
        gath[k].wait()

        buf = bufs[k % 2]
        half = (k % 2) * _HALF

        def row_add(r, _, buf=buf, half=half):
            for j in range(_VPR):
                plsc.addupdate(
                    buf.at[r, pl.ds(j * _LANES, _LANES)],
                    pe_v[half + r, pl.ds(j * _LANES, _LANES)],
                )
            return ()

        lax.fori_loop(0, _HALF, row_add, (), unroll=False)
        outc[k] = pltpu.async_copy(
            buf, out_hbm.at[k // 2, pl.ds(s0 + half, _HALF)], osems[k % 2])

    outc[_NCHUNK - 1].wait()


@jax.jit
def _embed(idx, table, pe):
    mesh = plsc.VectorSubcoreMesh(core_axis_name="c", subcore_axis_name="s")
    out = pl.kernel(
        _sc_body,
        out_type=jax.ShapeDtypeStruct((B, S, D_MODEL), jnp.float32),
        mesh=mesh,
        scratch_types=[
            pltpu.VMEM((B * _SPW,), jnp.int32),
            pltpu.VMEM((_SPW, D_MODEL), jnp.float32),
            pltpu.VMEM((_HALF, D_MODEL), jnp.float32),
            pltpu.VMEM((_HALF, D_MODEL), jnp.float32),
            pltpu.SemaphoreType.DMA,
            pltpu.SemaphoreType.DMA,
            pltpu.SemaphoreType.DMA,
            pltpu.SemaphoreType.DMA,
        ],
    )(table, idx, pe)
    return out


def kernel(x, table):
    return _embed(x.astype(jnp.int32), table, jnp.asarray(_PE))
